# Initial kernel scaffold; baseline (speedup 1.0000x reference)
import jax
import jax.numpy as jnp
from jax.experimental import pallas as pl


def kernel(user_code, item_code, user_occupation, item_timestamp_rank, deep_table, wide_table, wide_bias, W1, b1, g1, be1, W2, b2, g2, be2, W3, b3, g3, be3, W4, b4):
    raise NotImplementedError("write your pallas kernel here")


# speedup vs baseline: 3.8973x; 3.8973x over previous
import functools

import jax
import jax.numpy as jnp
from jax import lax
from jax.experimental import pallas as pl
from jax.experimental.pallas import tpu as pltpu
from jax.experimental.pallas import tpu_sc as plsc

_N_USERS = 1000000
_N_ITEMS = 1000000
_N_OCC = 100
_D = 32
_B = 1024
_NI = 200
_NTOK = _B * _NI
_NEX = 3 * _B

_NW = 32
_G = 128
_NG = 52
_SB = 26
_ROWS_W = _NG * _G
_NPAD = _NW * _ROWS_W

_BT = 64
_TOK = _BT * _NI
_NT = _B // _BT


def _sc_gather(deep_hbm, w16_hbm, idx_hbm, hi_hbm, lo_hbm,
               deep_out, wide_out,
               idx_v, hi_v, lo_v, rows_v, w16_v, wout_v, sem_g0, sem_g1):
    wid = lax.axis_index("s") * 2 + lax.axis_index("c")
    pltpu.sync_copy(idx_hbm.at[wid], idx_v)
    pltpu.sync_copy(hi_hbm.at[wid], hi_v)
    pltpu.sync_copy(lo_hbm.at[wid], lo_v)
    obase = wid * _ROWS_W
    semg = (sem_g0, sem_g1)

    def fire(k):
        par = k % 2
        return [
            pltpu.async_copy(deep_hbm.at[idx_v.at[k]],
                             rows_v.at[pl.ds(par * _G, _G)], semg[par]),
            pltpu.async_copy(w16_hbm.at[hi_v.at[k]],
                             w16_v.at[pl.ds(par * _G, _G)], semg[par]),
        ]

    prev = fire(0)
    for k in range(_NG):
        nxt = fire(k + 1) if k + 1 < _NG else None
        for h in prev:
            h.wait()
        par = k % 2
        pltpu.sync_copy(rows_v.at[pl.ds(par * _G, _G)],
                        deep_out.at[pl.ds(obase + k * _G, _G)])
        for c in range(8):
            rowv = jnp.arange(16, dtype=jnp.int32) + (par * _G + c * 16)
            lanev = lo_v[k, pl.ds(c * 16, 16)]
            wout_v[pl.ds(k * _G + c * 16, 16)] = plsc.load_gather(
                w16_v, [rowv, lanev])
        prev = nxt
    pltpu.sync_copy(wout_v, wide_out.at[pl.ds(obase, _ROWS_W)])


def _sc_gather_call(**kw):
    return functools.partial(
        pl.kernel,
        mesh=plsc.VectorSubcoreMesh(core_axis_name="c", subcore_axis_name="s",
                                    num_cores=2, num_subcores=16),
        scratch_types=[
            pltpu.VMEM((_NG, _G), jnp.int32),
            pltpu.VMEM((_NG, _G), jnp.int32),
            pltpu.VMEM((_NG, _G), jnp.int32),
            pltpu.VMEM((2 * _G, _D), jnp.float32),
            pltpu.VMEM((2 * _G, 16), jnp.float32),
            pltpu.VMEM((_ROWS_W,), jnp.float32),
            pltpu.SemaphoreType.DMA,
            pltpu.SemaphoreType.DMA,
        ],
        compiler_params=pltpu.CompilerParams(use_tc_tiling_on_sc=False,
                                             needs_layout_passes=False),
        **kw,
    )


def _mlp_kernel(item_ref, iwide_ref, exd_ref, exw_ref,
                w1a_ref, w1b_ref, b1_ref, g1_ref, be1_ref,
                w2_ref, b2_ref, g2_ref, be2_ref,
                w3_ref, b3_ref, g3_ref, be3_ref,
                w4_ref, b4_ref, wbias_ref,
                out_ref,
                s1_ref, q1_ref, s2_ref, q2_ref, s3_ref, q3_ref):
    p = pl.program_id(0)
    t = pl.program_id(1)
    n = jnp.float32(_NTOK)

    @pl.when((p == 0) & (t == 0))
    def _init():
        s1_ref[...] = jnp.zeros_like(s1_ref)
        q1_ref[...] = jnp.zeros_like(q1_ref)
        s2_ref[...] = jnp.zeros_like(s2_ref)
        q2_ref[...] = jnp.zeros_like(q2_ref)
        s3_ref[...] = jnp.zeros_like(s3_ref)
        q3_ref[...] = jnp.zeros_like(q3_ref)

    item = item_ref[...].reshape(_TOK, _D)
    exd = exd_ref[pl.ds(t * _BT, _BT), :]
    exc = jnp.dot(exd, w1b_ref[...],
                  preferred_element_type=jnp.float32) + b1_ref[...]
    h1 = jnp.dot(item, w1a_ref[...],
                 preferred_element_type=jnp.float32).reshape(_BT, _NI, 4 * _D)
    h1 = (h1 + exc[:, None, :]).reshape(_TOK, 4 * _D)

    def _bn_relu(h, s_ref, q_ref, g_ref, be_ref):
        mean = s_ref[...] / n
        var = q_ref[...] / n - mean * mean
        return jnp.maximum((h - mean) / jnp.sqrt(var + 1e-5) * g_ref[...]
                           + be_ref[...], 0.0)

    @pl.when(p == 0)
    def _p0():
        s1_ref[...] += jnp.sum(h1, axis=0, keepdims=True)
        q1_ref[...] += jnp.sum(h1 * h1, axis=0, keepdims=True)

    @pl.when(p == 1)
    def _p1():
        n1 = _bn_relu(h1, s1_ref, q1_ref, g1_ref, be1_ref)
        h2 = jnp.dot(n1, w2_ref[...],
                     preferred_element_type=jnp.float32) + b2_ref[...]
        s2_ref[...] += jnp.sum(h2, axis=0, keepdims=True)
        q2_ref[...] += jnp.sum(h2 * h2, axis=0, keepdims=True)

    @pl.when(p == 2)
    def _p2():
        n1 = _bn_relu(h1, s1_ref, q1_ref, g1_ref, be1_ref)
        h2 = jnp.dot(n1, w2_ref[...],
                     preferred_element_type=jnp.float32) + b2_ref[...]
        n2 = _bn_relu(h2, s2_ref, q2_ref, g2_ref, be2_ref)
        h3 = jnp.dot(n2, w3_ref[...],
                     preferred_element_type=jnp.float32) + b3_ref[...]
        s3_ref[...] += jnp.sum(h3, axis=0, keepdims=True)
        q3_ref[...] += jnp.sum(h3 * h3, axis=0, keepdims=True)

    @pl.when(p == 3)
    def _p3():
        n1 = _bn_relu(h1, s1_ref, q1_ref, g1_ref, be1_ref)
        h2 = jnp.dot(n1, w2_ref[...],
                     preferred_element_type=jnp.float32) + b2_ref[...]
        n2 = _bn_relu(h2, s2_ref, q2_ref, g2_ref, be2_ref)
        h3 = jnp.dot(n2, w3_ref[...],
                     preferred_element_type=jnp.float32) + b3_ref[...]
        n3 = _bn_relu(h3, s3_ref, q3_ref, g3_ref, be3_ref)
        h4 = jnp.sum(n3.reshape(_BT, _NI, _D)
                     * w4_ref[...].reshape(1, 1, _D), axis=-1)
        wsum = jnp.sum(exw_ref[pl.ds(t * _BT, _BT), :], axis=1, keepdims=True)
        out_ref[...] = (h4 + b4_ref[...] + iwide_ref[...] + wsum
                        + wbias_ref[...])


def _mlp_call(item_g, iwide_g, exd_g, exw_g, w1a, w1b, b1, g1, be1,
              w2, b2, g2, be2, w3, b3, g3, be3, w4, b4, wbias):
    full = lambda shape: pl.BlockSpec(shape, lambda p, t: (0,) * len(shape))
    return pl.pallas_call(
        _mlp_kernel,
        grid=(4, _NT),
        in_specs=[
            pl.BlockSpec((_BT, _NI, _D), lambda p, t: (t, 0, 0)),
            pl.BlockSpec((_BT, _NI), lambda p, t: (t, 0)),
            full((_B, 3 * _D)),
            full((_B, 3)),
            full((_D, 4 * _D)),
            full((3 * _D, 4 * _D)),
            full((1, 4 * _D)),
            full((1, 4 * _D)),
            full((1, 4 * _D)),
            full((4 * _D, 2 * _D)),
            full((1, 2 * _D)),
            full((1, 2 * _D)),
            full((1, 2 * _D)),
            full((2 * _D, _D)),
            full((1, _D)),
            full((1, _D)),
            full((1, _D)),
            full((_D, 1)),
            full((1, 1)),
            full((1, 1)),
        ],
        out_specs=pl.BlockSpec((_BT, _NI), lambda p, t: (t, 0)),
        scratch_shapes=[
            pltpu.VMEM((1, 4 * _D), jnp.float32),
            pltpu.VMEM((1, 4 * _D), jnp.float32),
            pltpu.VMEM((1, 2 * _D), jnp.float32),
            pltpu.VMEM((1, 2 * _D), jnp.float32),
            pltpu.VMEM((1, _D), jnp.float32),
            pltpu.VMEM((1, _D), jnp.float32),
        ],
        out_shape=jax.ShapeDtypeStruct((_B, _NI), jnp.float32),
        compiler_params=pltpu.CompilerParams(
            dimension_semantics=("arbitrary", "arbitrary")),
    )(item_g, iwide_g, exd_g, exw_g, w1a, w1b, b1, g1, be1,
      w2, b2, g2, be2, w3, b3, g3, be3, w4, b4, wbias)


def kernel(user_code, item_code, user_occupation, item_timestamp_rank,
           deep_table, wide_table, wide_bias,
           W1, b1, g1, be1, W2, b2, g2, be2, W3, b3, g3, be3, W4, b4):
    item_idx = (item_code + _N_USERS).astype(jnp.int32).reshape(-1)
    ex_idx = jnp.stack(
        [user_code.astype(jnp.int32),
         (user_occupation + (_N_USERS + _N_ITEMS)).astype(jnp.int32),
         (item_timestamp_rank + (_N_USERS + _N_ITEMS + _N_OCC)).astype(jnp.int32)],
        axis=1).reshape(-1)
    idx_all = jnp.concatenate(
        [item_idx, ex_idx,
         jnp.zeros((_NPAD - _NTOK - _NEX,), jnp.int32)]).reshape(_NW, _NG, _G)
    hi_all = idx_all >> 4
    lo_all = idx_all & 15
    wtab16 = jnp.concatenate(
        [wide_table.reshape(-1),
         jnp.zeros(((-wide_table.shape[0]) % 16,), jnp.float32)]).reshape(-1, 16)

    deep_g, wide_g = _sc_gather_call(
        out_type=[jax.ShapeDtypeStruct((_NPAD, _D), jnp.float32),
                  jax.ShapeDtypeStruct((_NPAD,), jnp.float32)],
    )(_sc_gather)(deep_table, wtab16, idx_all, hi_all, lo_all)

    item_g = deep_g[:_NTOK].reshape(_B, _NI, _D)
    exd_g = deep_g[_NTOK:_NTOK + _NEX].reshape(_B, 3 * _D)
    iwide_g = wide_g[:_NTOK].reshape(_B, _NI)
    exw_g = wide_g[_NTOK:_NTOK + _NEX].reshape(_B, 3)

    f32 = lambda x: x.astype(jnp.float32)
    out = _mlp_call(
        item_g, iwide_g, exd_g, exw_g,
        f32(W1[:_D]), f32(W1[_D:]), f32(b1).reshape(1, -1),
        f32(g1).reshape(1, -1), f32(be1).reshape(1, -1),
        f32(W2), f32(b2).reshape(1, -1), f32(g2).reshape(1, -1),
        f32(be2).reshape(1, -1),
        f32(W3), f32(b3).reshape(1, -1), f32(g3).reshape(1, -1),
        f32(be3).reshape(1, -1),
        f32(W4), f32(b4).reshape(1, 1), f32(wide_bias).reshape(1, 1))
    return out


# speedup vs baseline: 3.9220x; 1.0063x over previous
import functools

import jax
import jax.numpy as jnp
from jax import lax
from jax.experimental import pallas as pl
from jax.experimental.pallas import tpu as pltpu
from jax.experimental.pallas import tpu_sc as plsc

_N_USERS = 1000000
_N_ITEMS = 1000000
_N_OCC = 100
_D = 32
_B = 1024
_NI = 200
_NTOK = _B * _NI
_NEX = 3 * _B

_NW = 32
_G = 128
_NG = 52
_SB = 26
_ROWS_W = _NG * _G
_NPAD = _NW * _ROWS_W

_BT = 64
_TOK = _BT * _NI
_NT = _B // _BT


def _sc_gather(deep_hbm, w16_hbm, idx_hbm, hi_hbm, lo_hbm,
               deep_out, wide_out,
               idx_v, hi_v, lo_v, rows_v, w16_v, wout_v, sem_g0, sem_g1):
    wid = lax.axis_index("s") * 2 + lax.axis_index("c")
    pltpu.sync_copy(idx_hbm.at[wid], idx_v)
    pltpu.sync_copy(hi_hbm.at[wid], hi_v)
    pltpu.sync_copy(lo_hbm.at[wid], lo_v)
    obase = wid * _ROWS_W
    semg = (sem_g0, sem_g1)

    def fire(k):
        par = k % 2
        return [
            pltpu.async_copy(deep_hbm.at[idx_v.at[k]],
                             rows_v.at[pl.ds(par * _G, _G)], semg[par]),
            pltpu.async_copy(w16_hbm.at[hi_v.at[k]],
                             w16_v.at[pl.ds(par * _G, _G)], semg[par]),
        ]

    prev = fire(0)
    for k in range(_NG):
        nxt = fire(k + 1) if k + 1 < _NG else None
        for h in prev:
            h.wait()
        par = k % 2
        pltpu.sync_copy(rows_v.at[pl.ds(par * _G, _G)],
                        deep_out.at[pl.ds(obase + k * _G, _G)])
        for c in range(8):
            rowv = jnp.arange(16, dtype=jnp.int32) + (par * _G + c * 16)
            lanev = lo_v[k, pl.ds(c * 16, 16)]
            wout_v[pl.ds(k * _G + c * 16, 16)] = plsc.load_gather(
                w16_v, [rowv, lanev])
        prev = nxt
    pltpu.sync_copy(wout_v, wide_out.at[pl.ds(obase, _ROWS_W)])


def _sc_gather_call(**kw):
    return functools.partial(
        pl.kernel,
        mesh=plsc.VectorSubcoreMesh(core_axis_name="c", subcore_axis_name="s",
                                    num_cores=2, num_subcores=16),
        scratch_types=[
            pltpu.VMEM((_NG, _G), jnp.int32),
            pltpu.VMEM((_NG, _G), jnp.int32),
            pltpu.VMEM((_NG, _G), jnp.int32),
            pltpu.VMEM((2 * _G, _D), jnp.float32),
            pltpu.VMEM((2 * _G, 16), jnp.float32),
            pltpu.VMEM((_ROWS_W,), jnp.float32),
            pltpu.SemaphoreType.DMA,
            pltpu.SemaphoreType.DMA,
        ],
        compiler_params=pltpu.CompilerParams(use_tc_tiling_on_sc=False,
                                             needs_layout_passes=False),
        **kw,
    )


_ARB = pltpu.CompilerParams(dimension_semantics=("arbitrary",))
_F32 = jnp.float32


def _full(shape):
    return pl.BlockSpec(shape, lambda t: (0,) * len(shape))


def _scale_shift(sq_ref, g_ref, be_ref):
    n = jnp.float32(_NTOK)
    mean = sq_ref[0:1, :] / n
    var = sq_ref[1:2, :] / n - mean * mean
    scale = g_ref[...] / jnp.sqrt(var + 1e-5)
    return scale, be_ref[...] - mean * scale


def _accum_sums(t, h, sq_ref):
    @pl.when(t == 0)
    def _():
        sq_ref[...] = jnp.zeros_like(sq_ref)
    sq_ref[0:1, :] += jnp.sum(h, axis=0, keepdims=True)
    sq_ref[1:2, :] += jnp.sum(h * h, axis=0, keepdims=True)


def _p1_kernel(item_ref, exd_ref, w1a_ref, w1b_ref, b1_ref, h1_ref, sq1_ref):
    t = pl.program_id(0)
    item = item_ref[...].reshape(_TOK, _D)
    exd = exd_ref[pl.ds(t * _BT, _BT), :]
    exc = jnp.dot(exd, w1b_ref[...],
                  preferred_element_type=_F32) + b1_ref[...]
    h1 = jnp.dot(item, w1a_ref[...],
                 preferred_element_type=_F32).reshape(_BT, _NI, 4 * _D)
    h1 = (h1 + exc[:, None, :]).reshape(_TOK, 4 * _D)
    h1_ref[...] = h1
    _accum_sums(t, h1, sq1_ref)


def _mid_kernel(h_ref, sq_ref, g_ref, be_ref, w_ref, b_ref, ho_ref, sqo_ref):
    t = pl.program_id(0)
    scale, shift = _scale_shift(sq_ref, g_ref, be_ref)
    nh = jnp.maximum(h_ref[...] * scale + shift, 0.0)
    ho = jnp.dot(nh, w_ref[...], preferred_element_type=_F32) + b_ref[...]
    ho_ref[...] = ho
    _accum_sums(t, ho, sqo_ref)


def _p4_kernel(h_ref, sq_ref, g_ref, be_ref, w4_ref, b4_ref,
               iwide_ref, exw_ref, wb_ref, out_ref):
    t = pl.program_id(0)
    scale, shift = _scale_shift(sq_ref, g_ref, be_ref)
    n3 = jnp.maximum(h_ref[...] * scale + shift, 0.0)
    h4 = jnp.sum(n3.reshape(_BT, _NI, _D) * w4_ref[...].reshape(1, 1, _D),
                 axis=-1)
    wsum = jnp.sum(exw_ref[pl.ds(t * _BT, _BT), :], axis=1, keepdims=True)
    out_ref[...] = h4 + b4_ref[...] + iwide_ref[...] + wsum + wb_ref[...]


def _mlp_call(item_g, iwide_g, exd_g, exw_g, w1a, w1b, b1, g1, be1,
              w2, b2, g2, be2, w3, b3, g3, be3, w4, b4, wbias):
    tok_spec = lambda width: pl.BlockSpec((_TOK, width), lambda t: (t, 0))
    sq_spec = lambda width: pl.BlockSpec((2, width), lambda t: (0, 0))

    h1, sq1 = pl.pallas_call(
        _p1_kernel, grid=(_NT,),
        in_specs=[pl.BlockSpec((_BT, _NI, _D), lambda t: (t, 0, 0)),
                  _full((_B, 3 * _D)), _full((_D, 4 * _D)),
                  _full((3 * _D, 4 * _D)), _full((1, 4 * _D))],
        out_specs=[tok_spec(4 * _D), sq_spec(4 * _D)],
        out_shape=[jax.ShapeDtypeStruct((_NTOK, 4 * _D), _F32),
                   jax.ShapeDtypeStruct((2, 4 * _D), _F32)],
        compiler_params=_ARB,
    )(item_g, exd_g, w1a, w1b, b1)

    def mid(h, sq, g, be, w, b, fi, fo):
        return pl.pallas_call(
            _mid_kernel, grid=(_NT,),
            in_specs=[tok_spec(fi), sq_spec(fi), _full((1, fi)),
                      _full((1, fi)), _full((fi, fo)), _full((1, fo))],
            out_specs=[tok_spec(fo), sq_spec(fo)],
            out_shape=[jax.ShapeDtypeStruct((_NTOK, fo), _F32),
                       jax.ShapeDtypeStruct((2, fo), _F32)],
            compiler_params=_ARB,
        )(h, sq, g, be, w, b)

    h2, sq2 = mid(h1, sq1, g1, be1, w2, b2, 4 * _D, 2 * _D)
    h3, sq3 = mid(h2, sq2, g2, be2, w3, b3, 2 * _D, _D)

    return pl.pallas_call(
        _p4_kernel, grid=(_NT,),
        in_specs=[tok_spec(_D), sq_spec(_D), _full((1, _D)), _full((1, _D)),
                  _full((_D, 1)), _full((1, 1)),
                  pl.BlockSpec((_BT, _NI), lambda t: (t, 0)),
                  _full((_B, 3)), _full((1, 1))],
        out_specs=pl.BlockSpec((_BT, _NI), lambda t: (t, 0)),
        out_shape=jax.ShapeDtypeStruct((_B, _NI), _F32),
        compiler_params=_ARB,
    )(h3, sq3, g3, be3, w4, b4, iwide_g, exw_g, wbias)


def kernel(user_code, item_code, user_occupation, item_timestamp_rank,
           deep_table, wide_table, wide_bias,
           W1, b1, g1, be1, W2, b2, g2, be2, W3, b3, g3, be3, W4, b4):
    item_idx = (item_code + _N_USERS).astype(jnp.int32).reshape(-1)
    ex_idx = jnp.stack(
        [user_code.astype(jnp.int32),
         (user_occupation + (_N_USERS + _N_ITEMS)).astype(jnp.int32),
         (item_timestamp_rank + (_N_USERS + _N_ITEMS + _N_OCC)).astype(jnp.int32)],
        axis=1).reshape(-1)
    idx_all = jnp.concatenate(
        [item_idx, ex_idx,
         jnp.zeros((_NPAD - _NTOK - _NEX,), jnp.int32)]).reshape(_NW, _NG, _G)
    hi_all = idx_all >> 4
    lo_all = idx_all & 15
    wtab16 = jnp.concatenate(
        [wide_table.reshape(-1),
         jnp.zeros(((-wide_table.shape[0]) % 16,), jnp.float32)]).reshape(-1, 16)

    deep_g, wide_g = _sc_gather_call(
        out_type=[jax.ShapeDtypeStruct((_NPAD, _D), jnp.float32),
                  jax.ShapeDtypeStruct((_NPAD,), jnp.float32)],
    )(_sc_gather)(deep_table, wtab16, idx_all, hi_all, lo_all)

    item_g = deep_g[:_NTOK].reshape(_B, _NI, _D)
    exd_g = deep_g[_NTOK:_NTOK + _NEX].reshape(_B, 3 * _D)
    iwide_g = wide_g[:_NTOK].reshape(_B, _NI)
    exw_g = wide_g[_NTOK:_NTOK + _NEX].reshape(_B, 3)

    f32 = lambda x: x.astype(jnp.float32)
    out = _mlp_call(
        item_g, iwide_g, exd_g, exw_g,
        f32(W1[:_D]), f32(W1[_D:]), f32(b1).reshape(1, -1),
        f32(g1).reshape(1, -1), f32(be1).reshape(1, -1),
        f32(W2), f32(b2).reshape(1, -1), f32(g2).reshape(1, -1),
        f32(be2).reshape(1, -1),
        f32(W3), f32(b3).reshape(1, -1), f32(g3).reshape(1, -1),
        f32(be3).reshape(1, -1),
        f32(W4), f32(b4).reshape(1, 1), f32(wide_bias).reshape(1, 1))
    return out
